# baseline (speedup 1.0000x reference)
import functools

import jax
import jax.numpy as jnp
from jax import lax
from jax.experimental import pallas as pl
from jax.experimental.pallas import tpu as pltpu
from jax.experimental.pallas import tpu_sc as plsc

NC = 2
NS = 16
NW = NC * NS
NR = 4


def _lookup_kernel(n_bt, L, D, table_hbm, idx_hbm, out_hbm,
                   idxb, idxt, rows, outst, gsems, wsems):
    wid = lax.axis_index("s") * NC + lax.axis_index("c")
    nct = D // 8
    bt_per_w = n_bt // NW

    iota = lax.iota(jnp.int32, 16)
    iota_l = iota * L

    def gather(l, r):
        return pltpu.make_async_copy(
            table_hbm.at[idxt.at[pl.ds(l * 128, 128)]], rows.at[r],
            gsems.at[r])

    def write(l, nt, b, ct):
        return pltpu.make_async_copy(
            outst.at[b, pl.ds(ct * 1024, 1024)], out_hbm.at[l, ct, nt],
            wsems.at[b])

    for t in range(bt_per_w):
        nt = wid * bt_per_w + t
        pltpu.sync_copy(idx_hbm.at[pl.ds(nt * 128 * L, 128 * L)], idxb)

        @plsc.parallel_loop(0, L, step=1, unroll=2)
        def _(l):
            for g in range(8):
                v = plsc.load_gather(idxb, [iota_l + (g * 16 * L + l)])
                idxt[pl.ds(l * 128 + g * 16, 16)] = v

        for r in range(NR - 1):
            gather(r, r).start()

        def step(l, r, b):
            gather(l, r).wait()

            @pl.when(l + NR - 1 < L)
            def _():
                gather(l + NR - 1, (r + NR - 1) % NR).start()

            @pl.when(l >= NR)
            def _():
                for ct in range(nct):
                    write(l - NR, nt, b, ct).wait()

            @plsc.parallel_loop(0, D, step=1, unroll=4)
            def _(c):
                j = jnp.broadcast_to(c, (16,))
                for g in range(8):
                    v = plsc.load_gather(rows.at[r], [iota + g * 16, j])
                    outst[b, pl.ds(c * 128 + g * 16, 16)] = v

            for ct in range(nct):
                write(l, nt, b, ct).start()

        n_main = (L // NR) * NR

        def body(h, _):
            for d in range(NR):
                step(h * NR + d, d, d)
            return ()

        lax.fori_loop(0, L // NR, body, (), unroll=False)
        for l in range(n_main, L):
            step(l, l % NR, l % NR)

        for l in range(L - NR, L):
            for ct in range(nct):
                write(l, nt, l % NR, ct).wait()


def kernel(all_embs, inputs):
    V, D = all_embs.shape
    B, L = inputs.shape
    n_bt = B // 128
    assert B % 128 == 0 and n_bt % NW == 0 and D % 8 == 0 and L % 2 == 0

    idx_flat = inputs.reshape(B * L)

    mesh = plsc.VectorSubcoreMesh(core_axis_name="c", subcore_axis_name="s")
    out4 = pl.kernel(
        functools.partial(_lookup_kernel, n_bt, L, D),
        out_type=jax.ShapeDtypeStruct((L, D // 8, n_bt, 1024), jnp.float32),
        mesh=mesh,
        scratch_types=[
            pltpu.VMEM((128 * L,), jnp.int32),
            pltpu.VMEM((L * 128,), jnp.int32),
            pltpu.VMEM((NR, 128, D), jnp.float32),
            pltpu.VMEM((NR, (D // 8) * 1024), jnp.float32),
            pltpu.SemaphoreType.DMA((NR,)),
            pltpu.SemaphoreType.DMA((NR,)),
        ],
        compiler_params=pltpu.CompilerParams(use_tc_tiling_on_sc=False,
                                             needs_layout_passes=False),
    )(all_embs, idx_flat)

    out = out4.reshape(L, D // 8, n_bt, 8, 128)
    out = out.transpose(2, 4, 0, 1, 3)
    return out.reshape(B, L, D)
